# baseline (speedup 1.0000x reference)
import functools

import jax
import jax.numpy as jnp
from jax.experimental import pallas as pl
from jax.experimental.pallas import tpu as pltpu

_PHI = 0.61803398875
_TOP_K = 2


def _router_kernel(x_ref, wt_ref, b_ref, eft_ref, scores_ref, idx_ref):
    xt = x_ref[...]
    phase = jnp.dot(xt, wt_ref[...], preferred_element_type=jnp.float32)
    phase = phase + b_ref[...]

    ssq = jnp.sum(phase * phase, axis=-1, keepdims=True)
    nx = phase / jnp.maximum(jnp.sqrt(ssq), 1e-12)

    eft = eft_ref[...]
    essq = jnp.sum(eft * eft, axis=0, keepdims=True)
    ne = eft / jnp.maximum(jnp.sqrt(essq), 1e-12)

    res = jnp.dot(nx, ne, preferred_element_type=jnp.float32)
    act = jax.nn.sigmoid(10.0 * (res - _PHI))

    e_iota = jax.lax.broadcasted_iota(jnp.int32, act.shape, 1)
    big = jnp.int32(act.shape[-1])

    m1 = jnp.max(act, axis=-1, keepdims=True)
    i1 = jnp.min(jnp.where(act == m1, e_iota, big), axis=-1, keepdims=True)
    act2 = jnp.where(e_iota == i1, -1.0, act)
    m2 = jnp.max(act2, axis=-1, keepdims=True)
    i2 = jnp.min(jnp.where(act2 == m2, e_iota, big), axis=-1, keepdims=True)

    scores_ref[...] = jnp.concatenate([m1, m2], axis=-1)
    idx_ref[...] = jnp.concatenate([i1, i2], axis=-1)


@functools.partial(jax.jit, static_argnames=())
def kernel(x, W, b, expert_frequencies):
    B, T, D = x.shape
    E = expert_frequencies.shape[0]
    N = B * T
    M = 256

    x2 = x.reshape(N, D)
    wt = W.T
    b2 = b.reshape(1, D)
    eft = expert_frequencies.T

    grid = (N // M,)
    scores, idx = pl.pallas_call(
        _router_kernel,
        grid=grid,
        in_specs=[
            pl.BlockSpec((M, D), lambda i: (i, 0)),
            pl.BlockSpec((D, D), lambda i: (0, 0)),
            pl.BlockSpec((1, D), lambda i: (0, 0)),
            pl.BlockSpec((D, E), lambda i: (0, 0)),
        ],
        out_specs=[
            pl.BlockSpec((M, _TOP_K), lambda i: (i, 0)),
            pl.BlockSpec((M, _TOP_K), lambda i: (i, 0)),
        ],
        out_shape=[
            jax.ShapeDtypeStruct((N, _TOP_K), jnp.float32),
            jax.ShapeDtypeStruct((N, _TOP_K), jnp.int32),
        ],
        compiler_params=pltpu.CompilerParams(
            dimension_semantics=("arbitrary",),
        ),
    )(x2, wt, b2, eft)

    return scores.reshape(B, T, _TOP_K), idx.reshape(B, T, _TOP_K)


# speedup vs baseline: 1.0911x; 1.0911x over previous
import functools

import jax
import jax.numpy as jnp
from jax.experimental import pallas as pl
from jax.experimental.pallas import tpu as pltpu

_PHI = 0.61803398875
_TOP_K = 2


def _router_kernel(x_ref, wt_ref, b_ref, eft_ref, scores_ref, idx_ref, ne_ref):
    i = pl.program_id(0)
    m = x_ref.shape[0]

    @pl.when(i == 0)
    def _():
        eft = eft_ref[...]
        essq = jnp.sum(eft * eft, axis=0, keepdims=True)
        ne_ref[...] = eft / jnp.maximum(jnp.sqrt(essq), 1e-12)

    xt = x_ref[...]
    phase = jnp.dot(xt, wt_ref[...], preferred_element_type=jnp.float32)
    phase = phase + b_ref[...]

    ssq = jnp.sum(phase * phase, axis=-1, keepdims=True)
    nx = phase / jnp.maximum(jnp.sqrt(ssq), 1e-12)

    res = jnp.dot(nx, ne_ref[...], preferred_element_type=jnp.float32)
    act = jax.nn.sigmoid(10.0 * (res - _PHI))

    e_iota = jax.lax.broadcasted_iota(jnp.int32, act.shape, 1)
    big = jnp.int32(act.shape[-1])

    m1 = jnp.max(act, axis=-1, keepdims=True)
    i1 = jnp.min(jnp.where(act == m1, e_iota, big), axis=-1, keepdims=True)
    act2 = jnp.where(e_iota == i1, -1.0, act)
    m2 = jnp.max(act2, axis=-1, keepdims=True)
    i2 = jnp.min(jnp.where(act2 == m2, e_iota, big), axis=-1, keepdims=True)

    scores_ref[pl.ds(i * m, m), :] = jnp.concatenate([m1, m2], axis=-1)
    idx_ref[pl.ds(i * m, m), :] = jnp.concatenate([i1, i2], axis=-1)


@functools.partial(jax.jit, static_argnames=())
def kernel(x, W, b, expert_frequencies):
    B, T, D = x.shape
    E = expert_frequencies.shape[0]
    N = B * T
    M = 512

    x2 = x.reshape(N, D)
    wt = W.T
    b2 = b.reshape(1, D)
    eft = expert_frequencies.T

    grid = (N // M,)
    scores, idx = pl.pallas_call(
        _router_kernel,
        grid=grid,
        in_specs=[
            pl.BlockSpec((M, D), lambda i: (i, 0)),
            pl.BlockSpec((D, D), lambda i: (0, 0)),
            pl.BlockSpec((1, D), lambda i: (0, 0)),
            pl.BlockSpec((D, E), lambda i: (0, 0)),
        ],
        out_specs=[
            pl.BlockSpec((N, _TOP_K), lambda i: (0, 0)),
            pl.BlockSpec((N, _TOP_K), lambda i: (0, 0)),
        ],
        out_shape=[
            jax.ShapeDtypeStruct((N, _TOP_K), jnp.float32),
            jax.ShapeDtypeStruct((N, _TOP_K), jnp.int32),
        ],
        scratch_shapes=[pltpu.VMEM((D, E), jnp.float32)],
        compiler_params=pltpu.CompilerParams(
            dimension_semantics=("arbitrary",),
        ),
    )(x2, wt, b2, eft)

    return scores.reshape(B, T, _TOP_K), idx.reshape(B, T, _TOP_K)
